# Initial kernel scaffold; baseline (speedup 1.0000x reference)
import jax
import jax.numpy as jnp
from jax.experimental import pallas as pl


def kernel(x, position, pe):
    raise NotImplementedError("write your pallas kernel here")


# speedup vs baseline: 1.0249x; 1.0249x over previous
import functools

import jax
import jax.numpy as jnp
from jax import lax
from jax.experimental import pallas as pl
from jax.experimental.pallas import tpu as pltpu
from jax.experimental.pallas import tpu_sc as plsc

D_MODEL = 1024
LANES = 16
NUM_CORES = 2
NUM_SUBCORES = 16
NUM_WORKERS = NUM_CORES * NUM_SUBCORES
CHUNK = 32


def _sc_body(x_hbm, pos_hbm, pe_hbm, out_hbm, idx_v, rows_v, x_v, sem):
    wid = lax.axis_index("s") * NUM_CORES + lax.axis_index("c")
    b_total = x_hbm.shape[0]
    b_per_w = b_total // NUM_WORKERS
    n_chunks = b_per_w // CHUNK
    base = wid * b_per_w

    def chunk_step(ci, _):
        row0 = base + ci * CHUNK
        pltpu.sync_copy(pos_hbm.at[pl.ds(row0, CHUNK)], idx_v)
        gat = pltpu.async_copy(pe_hbm.at[idx_v], rows_v, sem)
        pltpu.sync_copy(x_hbm.at[pl.ds(row0, CHUNK)], x_v)
        gat.wait()

        def add_row(t, _):
            for j in range(D_MODEL // LANES):
                sl = pl.ds(j * LANES, LANES)
                x_v[t, sl] = x_v[t, sl] + rows_v[t, sl]
            return 0

        lax.fori_loop(0, CHUNK, add_row, 0)
        pltpu.sync_copy(x_v, out_hbm.at[pl.ds(row0, CHUNK)])
        return 0

    lax.fori_loop(0, n_chunks, chunk_step, 0)


@jax.jit
def _pe_add(x2d, pos1d, pe):
    mesh = plsc.VectorSubcoreMesh(core_axis_name="c", subcore_axis_name="s")
    kern = functools.partial(
        pl.kernel,
        mesh=mesh,
        out_type=jax.ShapeDtypeStruct(x2d.shape, jnp.float32),
        scratch_types=[
            pltpu.VMEM((CHUNK,), jnp.int32),
            pltpu.VMEM((CHUNK, D_MODEL), jnp.float32),
            pltpu.VMEM((CHUNK, D_MODEL), jnp.float32),
            pltpu.SemaphoreType.DMA,
        ],
    )(_sc_body)
    return kern(x2d, pos1d, pe)


def kernel(x, position, pe):
    b, s, d = x.shape
    x2d = x.reshape(b * s, d)
    pos1d = position.reshape(b * s).astype(jnp.int32)
    out = _pe_add(x2d, pos1d, pe)
    return out.reshape(b, s, d)


# speedup vs baseline: 1.9252x; 1.8784x over previous
import functools

import jax
import jax.numpy as jnp
from jax import lax
from jax.experimental import pallas as pl
from jax.experimental.pallas import tpu as pltpu
from jax.experimental.pallas import tpu_sc as plsc

D_MODEL = 1024
LANES = 16
NUM_CORES = 2
NUM_SUBCORES = 16
NUM_WORKERS = NUM_CORES * NUM_SUBCORES
B_TOTAL = 32768
B_PER_W = B_TOTAL // NUM_WORKERS
CHUNK = 8
NBUF = 4
LOOKAHEAD = 2
N_CHUNKS = B_PER_W // CHUNK
N_SUPER = N_CHUNKS // NBUF


def _sc_body(x_hbm, pos_hbm, pe_hbm, out_hbm,
             idx_all, pe_v, x_v, gat_sem, xin_sem, out_sem):
    wid = lax.axis_index("s") * NUM_CORES + lax.axis_index("c")
    base = wid * B_PER_W

    pltpu.sync_copy(pos_hbm.at[pl.ds(base, B_PER_W)], idx_all)

    def idx_ref(c):
        return idx_all.at[pl.ds(c * CHUNK, CHUNK)]

    def rows(c):
        return pl.ds(base + c * CHUNK, CHUNK)

    def issue_loads(c, s):
        pltpu.async_copy(pe_hbm.at[idx_ref(c)], pe_v.at[s], gat_sem.at[s])
        pltpu.async_copy(x_hbm.at[rows(c)], x_v.at[s], xin_sem.at[s])

    for c in range(LOOKAHEAD):
        issue_loads(c, c)

    def super_step(g, _):
        for b in range(NBUF):
            c = g * NBUF + b
            cl = c + LOOKAHEAD
            sl = (b + LOOKAHEAD) % NBUF

            @pl.when(cl < N_CHUNKS)
            def _():
                @pl.when(cl >= NBUF)
                def _():
                    pltpu.make_async_copy(
                        x_v.at[sl], out_hbm.at[rows(cl - NBUF)],
                        out_sem.at[sl]).wait()
                issue_loads(cl, sl)

            pltpu.make_async_copy(
                pe_hbm.at[idx_ref(c)], pe_v.at[b], gat_sem.at[b]).wait()
            pltpu.make_async_copy(
                x_hbm.at[rows(c)], x_v.at[b], xin_sem.at[b]).wait()

            def add_row(t, _):
                for j in range(D_MODEL // LANES):
                    d = pl.ds(j * LANES, LANES)
                    plsc.addupdate(x_v.at[b, t, d], pe_v[b, t, d])
                return 0

            lax.fori_loop(0, CHUNK, add_row, 0)
            pltpu.async_copy(x_v.at[b], out_hbm.at[rows(c)], out_sem.at[b])
        return 0

    lax.fori_loop(0, N_SUPER, super_step, 0)

    for b in range(NBUF):
        c = N_CHUNKS - NBUF + b
        pltpu.make_async_copy(
            x_v.at[b], out_hbm.at[rows(c)], out_sem.at[b]).wait()


@jax.jit
def _pe_add(x2d, pos1d, pe):
    mesh = plsc.VectorSubcoreMesh(core_axis_name="c", subcore_axis_name="s")
    kern = functools.partial(
        pl.kernel,
        mesh=mesh,
        out_type=jax.ShapeDtypeStruct(x2d.shape, jnp.float32),
        scratch_types=[
            pltpu.VMEM((B_PER_W,), jnp.int32),
            pltpu.VMEM((NBUF, CHUNK, D_MODEL), jnp.float32),
            pltpu.VMEM((NBUF, CHUNK, D_MODEL), jnp.float32),
            pltpu.SemaphoreType.DMA((NBUF,)),
            pltpu.SemaphoreType.DMA((NBUF,)),
            pltpu.SemaphoreType.DMA((NBUF,)),
        ],
    )(_sc_body)
    return kern(x2d, pos1d, pe)


def kernel(x, position, pe):
    b, s, d = x.shape
    x2d = x.reshape(b * s, d)
    pos1d = position.reshape(b * s).astype(jnp.int32)
    out = _pe_add(x2d, pos1d, pe)
    return out.reshape(b, s, d)
